# baseline (speedup 1.0000x reference)
import numpy as np
import jax
import jax.numpy as jnp
from jax.experimental import pallas as pl

_C, _T, _H, _W = 3, 32, 224, 224
_TS = _T // 4
_IDX = tuple(int(v) for v in np.linspace(0.0, _T - 1, _TS))


_WS = 2
_TB = _T // _WS
_SB = _TS // _WS
_LOCAL = tuple(
    tuple(v - k * _TB for v in _IDX if k * _TB <= v < (k + 1) * _TB)
    for k in range(_WS)
)


def _body(in_ref, fast_ref, slow_ref):
    x = in_ref[...]
    fast_ref[...] = x
    k = pl.program_id(0) % _WS
    for kk in range(_WS):
        @pl.when(k == kk)
        def _(kk=kk):
            for j in range(_SB):
                slow_ref[:, j] = x[:, _LOCAL[kk][j]]


_pack = pl.pallas_call(
    _body,
    grid=(_C * _WS,),
    out_shape=(
        jax.ShapeDtypeStruct((_C * _WS, _TB, _H, _W), jnp.float32),
        jax.ShapeDtypeStruct((_C * _WS, _SB, _H, _W), jnp.float32),
    ),
    in_specs=[pl.BlockSpec((1, _TB, _H, _W), lambda w: (w, 0, 0, 0))],
    out_specs=(
        pl.BlockSpec((1, _TB, _H, _W), lambda w: (w, 0, 0, 0)),
        pl.BlockSpec((1, _SB, _H, _W), lambda w: (w, 0, 0, 0)),
    ),
)


def kernel(frames):
    fast, slow = _pack(frames.reshape(_C * _WS, _TB, _H, _W))
    return (slow.reshape(_C, _TS, _H, _W), fast.reshape(_C, _T, _H, _W))


# speedup vs baseline: 1.0675x; 1.0675x over previous
import numpy as np
import jax
import jax.numpy as jnp
from jax.experimental import pallas as pl
from jax.experimental.pallas import tpu as pltpu

_C, _T, _H, _W = 3, 32, 224, 224
_TS = _T // 4
_N = _C * _T
_IDX = tuple(int(v) for v in np.linspace(0.0, _T - 1, _TS))

_CF = 4
_NCHUNK = _N // _CF
_NBUF = 8
_LAG = 2

_SLOW = []
for _g in range(_NCHUNK):
    _c, _gg = divmod(_g, _T // _CF)
    _sel = [_v for _v in _IDX if _gg * _CF <= _v < (_gg + 1) * _CF]
    assert len(_sel) == 1
    _SLOW.append((_sel[0] - _gg * _CF, _c * _TS + _IDX.index(_sel[0])))


def _body(in_hbm, fast_hbm, slow_hbm, bufs, in_sem, out_sem):
    def in_copy(g):
        return pltpu.make_async_copy(
            in_hbm.at[pl.ds(g * _CF, _CF)], bufs.at[g % _NBUF], in_sem)

    def out_copies(g):
        local, srow = _SLOW[g]
        return (
            pltpu.make_async_copy(
                bufs.at[g % _NBUF], fast_hbm.at[pl.ds(g * _CF, _CF)], out_sem),
            pltpu.make_async_copy(
                bufs.at[g % _NBUF, pl.ds(local, 1)],
                slow_hbm.at[pl.ds(srow, 1)], out_sem),
        )

    depth = _NBUF - _LAG
    outs_waited = 0
    for g in range(depth):
        in_copy(g).start()
    for g in range(_NCHUNK):
        in_copy(g).wait()
        for d in out_copies(g):
            d.start()
        nxt = g + depth
        if nxt < _NCHUNK:
            if g >= _LAG:
                for d in out_copies(outs_waited):
                    d.wait()
                outs_waited += 1
            in_copy(nxt).start()
    for g in range(outs_waited, _NCHUNK):
        for d in out_copies(g):
            d.wait()


_pack = pl.pallas_call(
    _body,
    out_shape=(
        jax.ShapeDtypeStruct((_N, _H, _W), jnp.float32),
        jax.ShapeDtypeStruct((_C * _TS, _H, _W), jnp.float32),
    ),
    in_specs=[pl.BlockSpec(memory_space=pl.ANY)],
    out_specs=(
        pl.BlockSpec(memory_space=pl.ANY),
        pl.BlockSpec(memory_space=pl.ANY),
    ),
    scratch_shapes=[
        pltpu.VMEM((_NBUF, _CF, _H, _W), jnp.float32),
        pltpu.SemaphoreType.DMA,
        pltpu.SemaphoreType.DMA,
    ],
)


def kernel(frames):
    fast, slow = _pack(frames.reshape(_N, _H, _W))
    return (slow.reshape(_C, _TS, _H, _W), fast.reshape(_C, _T, _H, _W))


# speedup vs baseline: 1.1245x; 1.0534x over previous
import numpy as np
import jax
import jax.numpy as jnp
from jax.experimental import pallas as pl
from jax.experimental.pallas import tpu as pltpu

_C, _T, _H, _W = 3, 32, 224, 224
_TS = _T // 4
_N = _C * _T
_IDX = tuple(int(v) for v in np.linspace(0.0, _T - 1, _TS))

_CF = 8
_NCHUNK = _N // _CF
_NBUF = 6
_LAG = 2

_SLOW = []
for _g in range(_NCHUNK):
    _c, _gg = divmod(_g, _T // _CF)
    _SLOW.append([
        (_v - _gg * _CF, _c * _TS + _IDX.index(_v))
        for _v in _IDX if _gg * _CF <= _v < (_gg + 1) * _CF])


def _body(in_hbm, fast_hbm, slow_hbm, bufs, in_sem, out_sem):
    def in_copy(g):
        return pltpu.make_async_copy(
            in_hbm.at[pl.ds(g * _CF, _CF)], bufs.at[g % _NBUF], in_sem)

    def out_copies(g):
        descs = [pltpu.make_async_copy(
            bufs.at[g % _NBUF], fast_hbm.at[pl.ds(g * _CF, _CF)], out_sem)]
        for local, srow in _SLOW[g]:
            descs.append(pltpu.make_async_copy(
                bufs.at[g % _NBUF, pl.ds(local, 1)],
                slow_hbm.at[pl.ds(srow, 1)], out_sem))
        return descs

    depth = _NBUF - _LAG
    outs_waited = 0
    for g in range(depth):
        in_copy(g).start()
    for g in range(_NCHUNK):
        in_copy(g).wait()
        for d in out_copies(g):
            d.start()
        nxt = g + depth
        if nxt < _NCHUNK:
            if g >= _LAG:
                for d in out_copies(outs_waited):
                    d.wait()
                outs_waited += 1
            in_copy(nxt).start()
    for g in range(outs_waited, _NCHUNK):
        for d in out_copies(g):
            d.wait()


_pack = pl.pallas_call(
    _body,
    out_shape=(
        jax.ShapeDtypeStruct((_N, _H, _W), jnp.float32),
        jax.ShapeDtypeStruct((_C * _TS, _H, _W), jnp.float32),
    ),
    in_specs=[pl.BlockSpec(memory_space=pl.ANY)],
    out_specs=(
        pl.BlockSpec(memory_space=pl.ANY),
        pl.BlockSpec(memory_space=pl.ANY),
    ),
    scratch_shapes=[
        pltpu.VMEM((_NBUF, _CF, _H, _W), jnp.float32),
        pltpu.SemaphoreType.DMA,
        pltpu.SemaphoreType.DMA,
    ],
)


def kernel(frames):
    fast, slow = _pack(frames.reshape(_N, _H, _W))
    return (slow.reshape(_C, _TS, _H, _W), fast.reshape(_C, _T, _H, _W))


# speedup vs baseline: 1.1341x; 1.0085x over previous
import numpy as np
import jax
import jax.numpy as jnp
from jax.experimental import pallas as pl
from jax.experimental.pallas import tpu as pltpu

_C, _T, _H, _W = 3, 32, 224, 224
_TS = _T // 4
_N = _C * _T
_IDX = tuple(int(v) for v in np.linspace(0.0, _T - 1, _TS))

_CF = 16
_NCHUNK = _N // _CF
_NBUF = 4
_LAG = 1

_SLOW = []
for _g in range(_NCHUNK):
    _c, _gg = divmod(_g, _T // _CF)
    _SLOW.append([
        (_v - _gg * _CF, _c * _TS + _IDX.index(_v))
        for _v in _IDX if _gg * _CF <= _v < (_gg + 1) * _CF])


def _body(in_hbm, fast_hbm, slow_hbm, bufs, in_sem, out_sem):
    def in_copy(g):
        return pltpu.make_async_copy(
            in_hbm.at[pl.ds(g * _CF, _CF)], bufs.at[g % _NBUF], in_sem)

    def out_copies(g):
        descs = [pltpu.make_async_copy(
            bufs.at[g % _NBUF], fast_hbm.at[pl.ds(g * _CF, _CF)], out_sem)]
        for local, srow in _SLOW[g]:
            descs.append(pltpu.make_async_copy(
                bufs.at[g % _NBUF, pl.ds(local, 1)],
                slow_hbm.at[pl.ds(srow, 1)], out_sem))
        return descs

    depth = _NBUF - _LAG
    outs_waited = 0
    for g in range(depth):
        in_copy(g).start()
    for g in range(_NCHUNK):
        in_copy(g).wait()
        for d in out_copies(g):
            d.start()
        nxt = g + depth
        if nxt < _NCHUNK:
            if g >= _LAG:
                for d in out_copies(outs_waited):
                    d.wait()
                outs_waited += 1
            in_copy(nxt).start()
    for g in range(outs_waited, _NCHUNK):
        for d in out_copies(g):
            d.wait()


_pack = pl.pallas_call(
    _body,
    out_shape=(
        jax.ShapeDtypeStruct((_N, _H, _W), jnp.float32),
        jax.ShapeDtypeStruct((_C * _TS, _H, _W), jnp.float32),
    ),
    in_specs=[pl.BlockSpec(memory_space=pl.ANY)],
    out_specs=(
        pl.BlockSpec(memory_space=pl.ANY),
        pl.BlockSpec(memory_space=pl.ANY),
    ),
    scratch_shapes=[
        pltpu.VMEM((_NBUF, _CF, _H, _W), jnp.float32),
        pltpu.SemaphoreType.DMA,
        pltpu.SemaphoreType.DMA,
    ],
)


def kernel(frames):
    fast, slow = _pack(frames.reshape(_N, _H, _W))
    return (slow.reshape(_C, _TS, _H, _W), fast.reshape(_C, _T, _H, _W))


# speedup vs baseline: 1.1574x; 1.0206x over previous
import numpy as np
import jax
import jax.numpy as jnp
from jax.experimental import pallas as pl

_C, _T, _H, _W = 3, 32, 224, 224
_TS = _T // 4
_IDX = tuple(int(v) for v in np.linspace(0.0, _T - 1, _TS))


def _body(in_ref, fast_ref, slow_ref):
    x = in_ref[...]
    fast_ref[...] = x
    for k, v in enumerate(_IDX):
        slow_ref[:, k] = x[:, v]


_pack = pl.pallas_call(
    _body,
    grid=(_C,),
    out_shape=(
        jax.ShapeDtypeStruct((_C, _T, _H, _W), jnp.float32),
        jax.ShapeDtypeStruct((_C, _TS, _H, _W), jnp.float32),
    ),
    in_specs=[pl.BlockSpec((1, _T, _H, _W), lambda c: (c, 0, 0, 0))],
    out_specs=(
        pl.BlockSpec((1, _T, _H, _W), lambda c: (c, 0, 0, 0)),
        pl.BlockSpec((1, _TS, _H, _W), lambda c: (c, 0, 0, 0)),
    ),
)


def kernel(frames):
    fast, slow = _pack(frames)
    return (slow, fast)
